# Initial kernel scaffold; baseline (speedup 1.0000x reference)
import jax
import jax.numpy as jnp
from jax.experimental import pallas as pl


def kernel(x, edge_index, W1, b1, W2, b2, We, be):
    raise NotImplementedError("write your pallas kernel here")


# speedup vs baseline: 20.4188x; 20.4188x over previous
import functools

import jax
import jax.numpy as jnp
from jax import lax
from jax.experimental import pallas as pl
from jax.experimental.pallas import tpu as pltpu
from jax.experimental.pallas import tpu_sc as plsc

N = 10000
E = 320000
D = 128
NC = 2
NS = 16
NW = NC * NS
EW = E // NW
B = 80
KB = EW // B
NACC = 10240
RPT = NACC // NS
NPAD = 80 * 128

_MESH = plsc.VectorSubcoreMesh(core_axis_name="c", subcore_axis_name="s")


def _wid():
    return lax.axis_index("s") * NC + lax.axis_index("c")


def _sc_deg_body(eb, zeros, out, didx, hist):
    wid = _wid()
    pltpu.sync_copy(eb.at[1, wid], didx)
    pltpu.sync_copy(zeros, hist)
    ones = jnp.full((16,), 1.0, jnp.float32)

    def step(j, carry):
        for k in range(B // 16):
            dv = didx[j, pl.ds(k * 16, 16)]
            row = lax.shift_right_logical(dv, 7)
            col = lax.bitwise_and(dv, 127)
            plsc.addupdate_scatter(hist, [row, col], ones)
        return carry

    lax.fori_loop(0, KB, step, 0)
    pltpu.sync_copy(hist, out.at[wid])


_sc_deg = pl.kernel(
    _sc_deg_body,
    out_type=jax.ShapeDtypeStruct((NW, 80, 128), jnp.float32),
    mesh=_MESH,
    compiler_params=pltpu.CompilerParams(needs_layout_passes=False),
    scratch_types=[
        pltpu.VMEM((KB, B), jnp.int32),
        pltpu.VMEM((80, 128), jnp.float32),
    ],
)


DH = D // 2


def _sc_msgpass_body(eb, ga, gb, zeros, outa, outb, sidx, didx, rows0, rows1,
                     acc, sem0, sem1):
    cid = lax.axis_index("c")
    sid = lax.axis_index("s")
    wid = sid * NC + cid
    pltpu.sync_copy(eb.at[0, wid], sidx)
    pltpu.sync_copy(eb.at[1, wid], didx)

    for g, out in ((ga, outa), (gb, outb)):
        pltpu.sync_copy(zeros, acc.at[pl.ds(sid * RPT, RPT)])
        plsc.subcore_barrier()

        def step(jj, carry):
            j = jj * 2
            c0 = pltpu.async_copy(g.at[sidx.at[j]], rows0, sem0)
            c1 = pltpu.async_copy(g.at[sidx.at[j + 1]], rows1, sem1)
            c0.wait()
            pltpu.sync_copy(rows0, acc.at[didx.at[j]], add=True)
            c1.wait()
            pltpu.sync_copy(rows1, acc.at[didx.at[j + 1]], add=True)
            return carry

        lax.fori_loop(0, KB // 2, step, 0)
        j = KB - 1
        pltpu.async_copy(g.at[sidx.at[j]], rows0, sem0).wait()
        pltpu.sync_copy(rows0, acc.at[didx.at[j]], add=True)
        plsc.subcore_barrier()
        pltpu.sync_copy(acc.at[pl.ds(sid * RPT, RPT)],
                        out.at[cid, pl.ds(sid * RPT, RPT)])
        plsc.subcore_barrier()


_sc_msgpass = pl.kernel(
    _sc_msgpass_body,
    out_type=(jax.ShapeDtypeStruct((NC, NACC, DH), jnp.float32),
              jax.ShapeDtypeStruct((NC, NACC, DH), jnp.float32)),
    mesh=_MESH,
    compiler_params=pltpu.CompilerParams(use_tc_tiling_on_sc=False),
    scratch_types=[
        pltpu.VMEM((KB, B), jnp.int32),
        pltpu.VMEM((KB, B), jnp.int32),
        pltpu.VMEM((B, DH), jnp.float32),
        pltpu.VMEM((B, DH), jnp.float32),
        pltpu.VMEM_SHARED((NACC, DH), jnp.float32),
        pltpu.SemaphoreType.DMA,
        pltpu.SemaphoreType.DMA,
    ],
)


def _sc_edgepred_body(eb, pa, pb, out, sidx, didx, pav, pbv, outv):
    wid = _wid()
    pltpu.sync_copy(eb.at[0, wid], sidx)
    pltpu.sync_copy(eb.at[1, wid], didx)
    pltpu.sync_copy(pa, pav)
    pltpu.sync_copy(pb, pbv)

    def step(j, carry):
        for k in range(B // 16):
            sv = sidx[j, pl.ds(k * 16, 16)]
            dv = didx[j, pl.ds(k * 16, 16)]
            r = plsc.load_gather(pav, [sv]) + plsc.load_gather(pbv, [dv])
            outv[j, pl.ds(k * 16, 16)] = r
        return carry

    lax.fori_loop(0, KB, step, 0)
    pltpu.sync_copy(outv, out.at[wid])


_sc_edgepred = pl.kernel(
    _sc_edgepred_body,
    out_type=jax.ShapeDtypeStruct((NW, KB, B), jnp.float32),
    mesh=_MESH,
    compiler_params=pltpu.CompilerParams(needs_layout_passes=False),
    scratch_types=[
        pltpu.VMEM((KB, B), jnp.int32),
        pltpu.VMEM((KB, B), jnp.int32),
        pltpu.VMEM((N,), jnp.float32),
        pltpu.VMEM((N,), jnp.float32),
        pltpu.VMEM((KB, B), jnp.float32),
    ],
)


def _tc_dinv_body(parts_ref, out_ref):
    deg = jnp.sum(parts_ref[...], axis=0) + 1.0
    out_ref[...] = lax.rsqrt(deg)


def _tc_dinv(parts):
    return pl.pallas_call(
        _tc_dinv_body,
        out_shape=jax.ShapeDtypeStruct((80, 128), jnp.float32),
    )(parts)


R = 2000


def _tc_mm_body(x_ref, w_ref, dinv_ref, h_ref, ga_ref, gb_ref):
    h = jnp.dot(x_ref[...], w_ref[...], preferred_element_type=jnp.float32)
    h_ref[...] = h
    g = h * dinv_ref[...]
    ga_ref[...] = g[:, :DH]
    gb_ref[...] = g[:, DH:]


def _tc_mm(x, w, dinv):
    return pl.pallas_call(
        _tc_mm_body,
        grid=(N // R,),
        in_specs=[
            pl.BlockSpec((R, D), lambda i: (i, 0)),
            pl.BlockSpec((D, D), lambda i: (0, 0)),
            pl.BlockSpec((R, 1), lambda i: (i, 0)),
        ],
        out_specs=[
            pl.BlockSpec((R, D), lambda i: (i, 0)),
            pl.BlockSpec((R, DH), lambda i: (i, 0)),
            pl.BlockSpec((R, DH), lambda i: (i, 0)),
        ],
        out_shape=[
            jax.ShapeDtypeStruct((N, D), jnp.float32),
            jax.ShapeDtypeStruct((N, DH), jnp.float32),
            jax.ShapeDtypeStruct((N, DH), jnp.float32),
        ],
    )(x, w, dinv)


def _tc_layer_body(pa_ref, pb_ref, dinv_ref, h_ref, b1_ref, w2_ref, t_ref,
                   g2a_ref, g2b_ref):
    s = jnp.concatenate([pa_ref[0] + pa_ref[1], pb_ref[0] + pb_ref[1]],
                        axis=1)
    dv = dinv_ref[...]
    agg = dv * s + (dv * dv) * h_ref[...] + b1_ref[...]
    h1 = jnp.maximum(agg, 0.0)
    t = jnp.dot(h1, w2_ref[...], preferred_element_type=jnp.float32)
    t_ref[...] = t
    g2 = t * dv
    g2a_ref[...] = g2[:, :DH]
    g2b_ref[...] = g2[:, DH:]


def _tc_layer(pa, pb, dinv, h, b1, w2):
    return pl.pallas_call(
        _tc_layer_body,
        grid=(N // R,),
        in_specs=[
            pl.BlockSpec((NC, R, DH), lambda i: (0, i, 0)),
            pl.BlockSpec((NC, R, DH), lambda i: (0, i, 0)),
            pl.BlockSpec((R, 1), lambda i: (i, 0)),
            pl.BlockSpec((R, D), lambda i: (i, 0)),
            pl.BlockSpec((1, D), lambda i: (0, 0)),
            pl.BlockSpec((D, D), lambda i: (0, 0)),
        ],
        out_specs=[
            pl.BlockSpec((R, D), lambda i: (i, 0)),
            pl.BlockSpec((R, DH), lambda i: (i, 0)),
            pl.BlockSpec((R, DH), lambda i: (i, 0)),
        ],
        out_shape=[
            jax.ShapeDtypeStruct((N, D), jnp.float32),
            jax.ShapeDtypeStruct((N, DH), jnp.float32),
            jax.ShapeDtypeStruct((N, DH), jnp.float32),
        ],
    )(pa, pb, dinv, h, b1, w2)


def _tc_final_body(qa_ref, qb_ref, dinv_ref, t_ref, b2_ref, wa_ref, wb_ref,
                   be_ref, pa_ref, pb_ref):
    s = jnp.concatenate([qa_ref[0] + qa_ref[1], qb_ref[0] + qb_ref[1]],
                        axis=1)
    dv = dinv_ref[...]
    h2 = dv * s + (dv * dv) * t_ref[...] + b2_ref[...]
    pa_ref[...] = (
        jnp.dot(h2, wa_ref[...], preferred_element_type=jnp.float32)
        + be_ref[0, 0])
    pb_ref[...] = jnp.dot(h2, wb_ref[...], preferred_element_type=jnp.float32)


def _tc_final(qa, qb, dinv, t, b2, wa, wb, be):
    return pl.pallas_call(
        _tc_final_body,
        grid=(N // R,),
        in_specs=[
            pl.BlockSpec((NC, R, DH), lambda i: (0, i, 0)),
            pl.BlockSpec((NC, R, DH), lambda i: (0, i, 0)),
            pl.BlockSpec((R, 1), lambda i: (i, 0)),
            pl.BlockSpec((R, D), lambda i: (i, 0)),
            pl.BlockSpec((1, D), lambda i: (0, 0)),
            pl.BlockSpec((D, 1), lambda i: (0, 0)),
            pl.BlockSpec((D, 1), lambda i: (0, 0)),
            pl.BlockSpec((1, 1), lambda i: (0, 0)),
        ],
        out_specs=[
            pl.BlockSpec((R, 1), lambda i: (i, 0)),
            pl.BlockSpec((R, 1), lambda i: (i, 0)),
        ],
        out_shape=[
            jax.ShapeDtypeStruct((N, 1), jnp.float32),
            jax.ShapeDtypeStruct((N, 1), jnp.float32),
        ],
    )(qa, qb, dinv, t, b2, wa, wb, be)


def kernel(x, edge_index, W1, b1, W2, b2, We, be):
    eb = edge_index.reshape(2, NW, KB, B)
    zeros128 = jnp.zeros((80, 128), jnp.float32)
    zeros64 = jnp.zeros((RPT, DH), jnp.float32)

    deg_parts = _sc_deg(eb, zeros128)
    dinv_pad = _tc_dinv(deg_parts)
    dinv = dinv_pad.reshape(NPAD, 1)[:N]

    h, g1a, g1b = _tc_mm(x, W1, dinv)
    acc1a, acc1b = _sc_msgpass(eb, g1a, g1b, zeros64)
    t, g2a, g2b = _tc_layer(acc1a, acc1b, dinv, h, b1.reshape(1, D), W2)
    acc2a, acc2b = _sc_msgpass(eb, g2a, g2b, zeros64)
    pa, pb = _tc_final(acc2a, acc2b, dinv, t, b2.reshape(1, D), We[:D],
                       We[D:], be.reshape(1, 1))

    out = _sc_edgepred(eb, pa.reshape(N), pb.reshape(N))
    return out.reshape(E, 1)


# speedup vs baseline: 29.4386x; 1.4417x over previous
import functools

import jax
import jax.numpy as jnp
from jax import lax
from jax.experimental import pallas as pl
from jax.experimental.pallas import tpu as pltpu
from jax.experimental.pallas import tpu_sc as plsc

N = 10000
E = 320000
D = 128
NC = 2
NS = 16
NW = NC * NS
EW = E // NW
B = 80
KB = EW // B
NACC = 10240
RPT = NACC // NS
NPAD = 80 * 128

_MESH = plsc.VectorSubcoreMesh(core_axis_name="c", subcore_axis_name="s")


def _wid():
    return lax.axis_index("s") * NC + lax.axis_index("c")


def _sc_deg_body(eb, zeros, out, didx, hist):
    wid = _wid()
    pltpu.sync_copy(eb.at[1, wid], didx)
    pltpu.sync_copy(zeros, hist)
    ones = jnp.full((16,), 1.0, jnp.float32)

    def step(j, carry):
        for k in range(B // 16):
            dv = didx[j, pl.ds(k * 16, 16)]
            row = lax.shift_right_logical(dv, 7)
            col = lax.bitwise_and(dv, 127)
            plsc.addupdate_scatter(hist, [row, col], ones)
        return carry

    lax.fori_loop(0, KB, step, 0)
    pltpu.sync_copy(hist, out.at[wid])


_sc_deg = pl.kernel(
    _sc_deg_body,
    out_type=jax.ShapeDtypeStruct((NW, 80, 128), jnp.float32),
    mesh=_MESH,
    compiler_params=pltpu.CompilerParams(needs_layout_passes=False),
    scratch_types=[
        pltpu.VMEM((KB, B), jnp.int32),
        pltpu.VMEM((80, 128), jnp.float32),
    ],
)


DH = D // 2


NBUF = 8


def _sc_msgpass_body(eb, ga, gb, zeros, outa, outb, sidx, didx, *rest):
    rows = rest[:NBUF]
    acc = rest[NBUF]
    gsem = rest[NBUF + 1:NBUF + 1 + NBUF]
    ssem = rest[NBUF + 1 + NBUF:]
    cid = lax.axis_index("c")
    sid = lax.axis_index("s")
    wid = sid * NC + cid
    pltpu.sync_copy(eb.at[0, wid], sidx)
    pltpu.sync_copy(eb.at[1, wid], didx)

    for g, out in ((ga, outa), (gb, outb)):
        pltpu.sync_copy(zeros, acc.at[pl.ds(sid * RPT, RPT)])
        plsc.subcore_barrier()

        for k in range(NBUF // 2):
            pltpu.async_copy(g.at[sidx.at[k]], rows[k], gsem[k])

        def step(jj, carry):
            for k in range(NBUF):
                j = jj * NBUF + k
                b2 = (k + NBUF // 2) % NBUF

                @pl.when(j < KB)
                def _():
                    pltpu.make_async_copy(
                        g.at[sidx.at[0]], rows[k], gsem[k]).wait()
                    pltpu.async_copy(rows[k], acc.at[didx.at[j]], ssem[k],
                                     add=True)

                jn = j + NBUF // 2

                @pl.when(jn < KB)
                def _():
                    @pl.when(j >= NBUF // 2)
                    def _():
                        pltpu.make_async_copy(
                            rows[b2], acc.at[didx.at[0]], ssem[b2]).wait()
                    pltpu.async_copy(g.at[sidx.at[jn]], rows[b2], gsem[b2])

            return carry

        lax.fori_loop(0, (KB + NBUF - 1) // NBUF, step, 0)
        for k in range(NBUF):
            pltpu.make_async_copy(rows[k], acc.at[didx.at[0]],
                                  ssem[k]).wait()
        plsc.subcore_barrier()
        pltpu.sync_copy(acc.at[pl.ds(sid * RPT, RPT)],
                        out.at[cid, pl.ds(sid * RPT, RPT)])
        plsc.subcore_barrier()


_sc_msgpass = pl.kernel(
    _sc_msgpass_body,
    out_type=(jax.ShapeDtypeStruct((NC, NACC, DH), jnp.float32),
              jax.ShapeDtypeStruct((NC, NACC, DH), jnp.float32)),
    mesh=_MESH,
    compiler_params=pltpu.CompilerParams(use_tc_tiling_on_sc=False),
    scratch_types=(
        [pltpu.VMEM((KB, B), jnp.int32),
         pltpu.VMEM((KB, B), jnp.int32)]
        + [pltpu.VMEM((B, DH), jnp.float32) for _ in range(NBUF)]
        + [pltpu.VMEM_SHARED((NACC, DH), jnp.float32)]
        + [pltpu.SemaphoreType.DMA for _ in range(2 * NBUF)]
    ),
)


def _sc_edgepred_body(eb, pa, pb, out, sidx, didx, pav, pbv, outv):
    wid = _wid()
    pltpu.sync_copy(eb.at[0, wid], sidx)
    pltpu.sync_copy(eb.at[1, wid], didx)
    pltpu.sync_copy(pa, pav)
    pltpu.sync_copy(pb, pbv)

    def step(j, carry):
        for k in range(B // 16):
            sv = sidx[j, pl.ds(k * 16, 16)]
            dv = didx[j, pl.ds(k * 16, 16)]
            r = plsc.load_gather(pav, [sv]) + plsc.load_gather(pbv, [dv])
            outv[j, pl.ds(k * 16, 16)] = r
        return carry

    lax.fori_loop(0, KB, step, 0)
    pltpu.sync_copy(outv, out.at[wid])


_sc_edgepred = pl.kernel(
    _sc_edgepred_body,
    out_type=jax.ShapeDtypeStruct((NW, KB, B), jnp.float32),
    mesh=_MESH,
    compiler_params=pltpu.CompilerParams(needs_layout_passes=False),
    scratch_types=[
        pltpu.VMEM((KB, B), jnp.int32),
        pltpu.VMEM((KB, B), jnp.int32),
        pltpu.VMEM((N,), jnp.float32),
        pltpu.VMEM((N,), jnp.float32),
        pltpu.VMEM((KB, B), jnp.float32),
    ],
)


def _tc_dinv_body(parts_ref, out_ref):
    deg = jnp.sum(parts_ref[...], axis=0) + 1.0
    out_ref[...] = lax.rsqrt(deg)


def _tc_dinv(parts):
    return pl.pallas_call(
        _tc_dinv_body,
        out_shape=jax.ShapeDtypeStruct((80, 128), jnp.float32),
    )(parts)


R = 2000


def _tc_mm_body(x_ref, w_ref, dinv_ref, h_ref, ga_ref, gb_ref):
    h = jnp.dot(x_ref[...], w_ref[...], preferred_element_type=jnp.float32)
    h_ref[...] = h
    g = h * dinv_ref[...]
    ga_ref[...] = g[:, :DH]
    gb_ref[...] = g[:, DH:]


def _tc_mm(x, w, dinv):
    return pl.pallas_call(
        _tc_mm_body,
        grid=(N // R,),
        in_specs=[
            pl.BlockSpec((R, D), lambda i: (i, 0)),
            pl.BlockSpec((D, D), lambda i: (0, 0)),
            pl.BlockSpec((R, 1), lambda i: (i, 0)),
        ],
        out_specs=[
            pl.BlockSpec((R, D), lambda i: (i, 0)),
            pl.BlockSpec((R, DH), lambda i: (i, 0)),
            pl.BlockSpec((R, DH), lambda i: (i, 0)),
        ],
        out_shape=[
            jax.ShapeDtypeStruct((N, D), jnp.float32),
            jax.ShapeDtypeStruct((N, DH), jnp.float32),
            jax.ShapeDtypeStruct((N, DH), jnp.float32),
        ],
    )(x, w, dinv)


def _tc_layer_body(pa_ref, pb_ref, dinv_ref, h_ref, b1_ref, w2_ref, t_ref,
                   g2a_ref, g2b_ref):
    s = jnp.concatenate([pa_ref[0] + pa_ref[1], pb_ref[0] + pb_ref[1]],
                        axis=1)
    dv = dinv_ref[...]
    agg = dv * s + (dv * dv) * h_ref[...] + b1_ref[...]
    h1 = jnp.maximum(agg, 0.0)
    t = jnp.dot(h1, w2_ref[...], preferred_element_type=jnp.float32)
    t_ref[...] = t
    g2 = t * dv
    g2a_ref[...] = g2[:, :DH]
    g2b_ref[...] = g2[:, DH:]


def _tc_layer(pa, pb, dinv, h, b1, w2):
    return pl.pallas_call(
        _tc_layer_body,
        grid=(N // R,),
        in_specs=[
            pl.BlockSpec((NC, R, DH), lambda i: (0, i, 0)),
            pl.BlockSpec((NC, R, DH), lambda i: (0, i, 0)),
            pl.BlockSpec((R, 1), lambda i: (i, 0)),
            pl.BlockSpec((R, D), lambda i: (i, 0)),
            pl.BlockSpec((1, D), lambda i: (0, 0)),
            pl.BlockSpec((D, D), lambda i: (0, 0)),
        ],
        out_specs=[
            pl.BlockSpec((R, D), lambda i: (i, 0)),
            pl.BlockSpec((R, DH), lambda i: (i, 0)),
            pl.BlockSpec((R, DH), lambda i: (i, 0)),
        ],
        out_shape=[
            jax.ShapeDtypeStruct((N, D), jnp.float32),
            jax.ShapeDtypeStruct((N, DH), jnp.float32),
            jax.ShapeDtypeStruct((N, DH), jnp.float32),
        ],
    )(pa, pb, dinv, h, b1, w2)


def _tc_final_body(qa_ref, qb_ref, dinv_ref, t_ref, b2_ref, wa_ref, wb_ref,
                   be_ref, pa_ref, pb_ref):
    s = jnp.concatenate([qa_ref[0] + qa_ref[1], qb_ref[0] + qb_ref[1]],
                        axis=1)
    dv = dinv_ref[...]
    h2 = dv * s + (dv * dv) * t_ref[...] + b2_ref[...]
    pa_ref[...] = (
        jnp.dot(h2, wa_ref[...], preferred_element_type=jnp.float32)
        + be_ref[0, 0])
    pb_ref[...] = jnp.dot(h2, wb_ref[...], preferred_element_type=jnp.float32)


def _tc_final(qa, qb, dinv, t, b2, wa, wb, be):
    return pl.pallas_call(
        _tc_final_body,
        grid=(N // R,),
        in_specs=[
            pl.BlockSpec((NC, R, DH), lambda i: (0, i, 0)),
            pl.BlockSpec((NC, R, DH), lambda i: (0, i, 0)),
            pl.BlockSpec((R, 1), lambda i: (i, 0)),
            pl.BlockSpec((R, D), lambda i: (i, 0)),
            pl.BlockSpec((1, D), lambda i: (0, 0)),
            pl.BlockSpec((D, 1), lambda i: (0, 0)),
            pl.BlockSpec((D, 1), lambda i: (0, 0)),
            pl.BlockSpec((1, 1), lambda i: (0, 0)),
        ],
        out_specs=[
            pl.BlockSpec((R, 1), lambda i: (i, 0)),
            pl.BlockSpec((R, 1), lambda i: (i, 0)),
        ],
        out_shape=[
            jax.ShapeDtypeStruct((N, 1), jnp.float32),
            jax.ShapeDtypeStruct((N, 1), jnp.float32),
        ],
    )(qa, qb, dinv, t, b2, wa, wb, be)


def kernel(x, edge_index, W1, b1, W2, b2, We, be):
    eb = edge_index.reshape(2, NW, KB, B)
    zeros128 = jnp.zeros((80, 128), jnp.float32)
    zeros64 = jnp.zeros((RPT, DH), jnp.float32)

    deg_parts = _sc_deg(eb, zeros128)
    dinv_pad = _tc_dinv(deg_parts)
    dinv = dinv_pad.reshape(NPAD, 1)[:N]

    h, g1a, g1b = _tc_mm(x, W1, dinv)
    acc1a, acc1b = _sc_msgpass(eb, g1a, g1b, zeros64)
    t, g2a, g2b = _tc_layer(acc1a, acc1b, dinv, h, b1.reshape(1, D), W2)
    acc2a, acc2b = _sc_msgpass(eb, g2a, g2b, zeros64)
    pa, pb = _tc_final(acc2a, acc2b, dinv, t, b2.reshape(1, D), We[:D],
                       We[D:], be.reshape(1, 1))

    out = _sc_edgepred(eb, pa.reshape(N), pb.reshape(N))
    return out.reshape(E, 1)


# speedup vs baseline: 32.7288x; 1.1118x over previous
import functools

import jax
import jax.numpy as jnp
from jax import lax
from jax.experimental import pallas as pl
from jax.experimental.pallas import tpu as pltpu
from jax.experimental.pallas import tpu_sc as plsc

N = 10000
E = 320000
D = 128
NC = 2
NS = 16
NW = NC * NS
EW = E // NW
B = 80
KB = EW // B
NACC = 10240
RPT = NACC // NS
NPAD = 80 * 128

_MESH = plsc.VectorSubcoreMesh(core_axis_name="c", subcore_axis_name="s")


def _wid():
    return lax.axis_index("s") * NC + lax.axis_index("c")


def _sc_deg_body(eb, zeros, out, didx, hist):
    wid = _wid()
    pltpu.sync_copy(eb.at[1, wid], didx)
    pltpu.sync_copy(zeros, hist)
    ones = jnp.full((16,), 1.0, jnp.float32)

    def step(j, carry):
        for k in range(B // 16):
            dv = didx[j, pl.ds(k * 16, 16)]
            row = lax.shift_right_logical(dv, 7)
            col = lax.bitwise_and(dv, 127)
            plsc.addupdate_scatter(hist, [row, col], ones)
        return carry

    lax.fori_loop(0, KB, step, 0)
    pltpu.sync_copy(hist, out.at[wid])


_sc_deg = pl.kernel(
    _sc_deg_body,
    out_type=jax.ShapeDtypeStruct((NW, 80, 128), jnp.float32),
    mesh=_MESH,
    compiler_params=pltpu.CompilerParams(needs_layout_passes=False),
    scratch_types=[
        pltpu.VMEM((KB, B), jnp.int32),
        pltpu.VMEM((80, 128), jnp.float32),
    ],
)


DH = D // 2


NBUF = 8


def _sc_msgpass_body(eb, ga, gb, zeros, out, sidx, didx, *rest):
    rows = rest[:NBUF]
    acc = rest[NBUF]
    gsem = rest[NBUF + 1:NBUF + 1 + NBUF]
    ssem = rest[NBUF + 1 + NBUF:]
    cid = lax.axis_index("c")
    sid = lax.axis_index("s")
    wid = sid * NC + cid
    pltpu.sync_copy(eb.at[0, wid], sidx)
    pltpu.sync_copy(eb.at[1, wid], didx)

    for half, g in enumerate((ga, gb)):
        pltpu.sync_copy(zeros, acc.at[pl.ds(sid * RPT, RPT)])
        plsc.subcore_barrier()

        for k in range(NBUF // 2):
            pltpu.async_copy(g.at[sidx.at[k]], rows[k], gsem[k])

        def step(jj, carry):
            for k in range(NBUF):
                j = jj * NBUF + k
                b2 = (k + NBUF // 2) % NBUF

                @pl.when(j < KB)
                def _():
                    pltpu.make_async_copy(
                        g.at[sidx.at[0]], rows[k], gsem[k]).wait()
                    pltpu.async_copy(rows[k], acc.at[didx.at[j]], ssem[k],
                                     add=True)

                jn = j + NBUF // 2

                @pl.when(jn < KB)
                def _():
                    @pl.when(j >= NBUF // 2)
                    def _():
                        pltpu.make_async_copy(
                            rows[b2], acc.at[didx.at[0]], ssem[b2]).wait()
                    pltpu.async_copy(g.at[sidx.at[jn]], rows[b2], gsem[b2])

            return carry

        lax.fori_loop(0, (KB + NBUF - 1) // NBUF, step, 0)
        for k in range(NBUF):
            pltpu.make_async_copy(rows[k], acc.at[didx.at[0]],
                                  ssem[k]).wait()
        plsc.subcore_barrier()
        pltpu.sync_copy(acc.at[pl.ds(sid * RPT, RPT)],
                        out.at[cid, pl.ds(sid * RPT, RPT),
                               pl.ds(half * DH, DH)])
        plsc.subcore_barrier()


_sc_msgpass = pl.kernel(
    _sc_msgpass_body,
    out_type=jax.ShapeDtypeStruct((NC, NACC, D), jnp.float32),
    mesh=_MESH,
    compiler_params=pltpu.CompilerParams(use_tc_tiling_on_sc=False),
    scratch_types=(
        [pltpu.VMEM((KB, B), jnp.int32),
         pltpu.VMEM((KB, B), jnp.int32)]
        + [pltpu.VMEM((B, DH), jnp.float32) for _ in range(NBUF)]
        + [pltpu.VMEM_SHARED((NACC, DH), jnp.float32)]
        + [pltpu.SemaphoreType.DMA for _ in range(2 * NBUF)]
    ),
)


def _sc_edgepred_body(eb, pa, pb, out, sidx, didx, pav, pbv, outv):
    wid = _wid()
    pltpu.sync_copy(eb.at[0, wid], sidx)
    pltpu.sync_copy(eb.at[1, wid], didx)
    pltpu.sync_copy(pa, pav)
    pltpu.sync_copy(pb, pbv)

    def step(j, carry):
        for k in range(B // 16):
            sv = sidx[j, pl.ds(k * 16, 16)]
            dv = didx[j, pl.ds(k * 16, 16)]
            r = plsc.load_gather(pav, [sv]) + plsc.load_gather(pbv, [dv])
            outv[j, pl.ds(k * 16, 16)] = r
        return carry

    lax.fori_loop(0, KB, step, 0)
    pltpu.sync_copy(outv, out.at[wid])


_sc_edgepred = pl.kernel(
    _sc_edgepred_body,
    out_type=jax.ShapeDtypeStruct((NW, KB, B), jnp.float32),
    mesh=_MESH,
    compiler_params=pltpu.CompilerParams(needs_layout_passes=False),
    scratch_types=[
        pltpu.VMEM((KB, B), jnp.int32),
        pltpu.VMEM((KB, B), jnp.int32),
        pltpu.VMEM((N,), jnp.float32),
        pltpu.VMEM((N,), jnp.float32),
        pltpu.VMEM((KB, B), jnp.float32),
    ],
)


def _tc_dinv_body(parts_ref, out_ref):
    deg = jnp.sum(parts_ref[...], axis=0) + 1.0
    out_ref[...] = lax.rsqrt(deg)


def _tc_dinv(parts):
    return pl.pallas_call(
        _tc_dinv_body,
        out_shape=jax.ShapeDtypeStruct((80, 128), jnp.float32),
    )(parts)


R = 2000


def _tc_mm_body(x_ref, w_ref, dinv_ref, h_ref, ga_ref, gb_ref):
    h = jnp.dot(x_ref[...], w_ref[...], preferred_element_type=jnp.float32)
    h_ref[...] = h
    g = h * dinv_ref[...]
    ga_ref[...] = g[:, :DH]
    gb_ref[...] = g[:, DH:]


def _tc_mm(x, w, dinv):
    return pl.pallas_call(
        _tc_mm_body,
        grid=(N // R,),
        in_specs=[
            pl.BlockSpec((R, D), lambda i: (i, 0)),
            pl.BlockSpec((D, D), lambda i: (0, 0)),
            pl.BlockSpec((R, 1), lambda i: (i, 0)),
        ],
        out_specs=[
            pl.BlockSpec((R, D), lambda i: (i, 0)),
            pl.BlockSpec((R, DH), lambda i: (i, 0)),
            pl.BlockSpec((R, DH), lambda i: (i, 0)),
        ],
        out_shape=[
            jax.ShapeDtypeStruct((N, D), jnp.float32),
            jax.ShapeDtypeStruct((N, DH), jnp.float32),
            jax.ShapeDtypeStruct((N, DH), jnp.float32),
        ],
    )(x, w, dinv)


def _tc_layer_body(parts_ref, dinv_ref, h_ref, b1_ref, w2_ref, t_ref,
                   g2a_ref, g2b_ref):
    s = parts_ref[0] + parts_ref[1]
    dv = dinv_ref[...]
    agg = dv * s + (dv * dv) * h_ref[...] + b1_ref[...]
    h1 = jnp.maximum(agg, 0.0)
    t = jnp.dot(h1, w2_ref[...], preferred_element_type=jnp.float32)
    t_ref[...] = t
    g2 = t * dv
    g2a_ref[...] = g2[:, :DH]
    g2b_ref[...] = g2[:, DH:]


def _tc_layer(parts, dinv, h, b1, w2):
    return pl.pallas_call(
        _tc_layer_body,
        grid=(N // R,),
        in_specs=[
            pl.BlockSpec((NC, R, D), lambda i: (0, i, 0)),
            pl.BlockSpec((R, 1), lambda i: (i, 0)),
            pl.BlockSpec((R, D), lambda i: (i, 0)),
            pl.BlockSpec((1, D), lambda i: (0, 0)),
            pl.BlockSpec((D, D), lambda i: (0, 0)),
        ],
        out_specs=[
            pl.BlockSpec((R, D), lambda i: (i, 0)),
            pl.BlockSpec((R, DH), lambda i: (i, 0)),
            pl.BlockSpec((R, DH), lambda i: (i, 0)),
        ],
        out_shape=[
            jax.ShapeDtypeStruct((N, D), jnp.float32),
            jax.ShapeDtypeStruct((N, DH), jnp.float32),
            jax.ShapeDtypeStruct((N, DH), jnp.float32),
        ],
    )(parts, dinv, h, b1, w2)


def _tc_final_body(parts_ref, dinv_ref, t_ref, b2_ref, wa_ref, wb_ref,
                   be_ref, pa_ref, pb_ref):
    s = parts_ref[0] + parts_ref[1]
    dv = dinv_ref[...]
    h2 = dv * s + (dv * dv) * t_ref[...] + b2_ref[...]
    pa_ref[...] = jnp.sum(h2 * wa_ref[...], axis=1) + be_ref[0, 0]
    pb_ref[...] = jnp.sum(h2 * wb_ref[...], axis=1)


def _tc_final(parts, dinv, t, b2, wa, wb, be):
    return pl.pallas_call(
        _tc_final_body,
        grid=(1,),
        in_specs=[
            pl.BlockSpec((NC, N, D), lambda i: (0, 0, 0)),
            pl.BlockSpec((N, 1), lambda i: (0, 0)),
            pl.BlockSpec((N, D), lambda i: (0, 0)),
            pl.BlockSpec((1, D), lambda i: (0, 0)),
            pl.BlockSpec((1, D), lambda i: (0, 0)),
            pl.BlockSpec((1, D), lambda i: (0, 0)),
            pl.BlockSpec((1, 1), lambda i: (0, 0)),
        ],
        out_specs=[
            pl.BlockSpec((N,), lambda i: (0,)),
            pl.BlockSpec((N,), lambda i: (0,)),
        ],
        out_shape=[
            jax.ShapeDtypeStruct((N,), jnp.float32),
            jax.ShapeDtypeStruct((N,), jnp.float32),
        ],
    )(parts, dinv, t, b2, wa, wb, be)


def kernel(x, edge_index, W1, b1, W2, b2, We, be):
    eb = edge_index.reshape(2, NW, KB, B)
    zeros128 = jnp.zeros((80, 128), jnp.float32)
    zeros64 = jnp.zeros((RPT, DH), jnp.float32)

    deg_parts = _sc_deg(eb, zeros128)
    dinv_pad = _tc_dinv(deg_parts)
    dinv = dinv_pad.reshape(NPAD, 1)[:N]

    h, g1a, g1b = _tc_mm(x, W1, dinv)
    acc1 = _sc_msgpass(eb, g1a, g1b, zeros64)
    t, g2a, g2b = _tc_layer(acc1, dinv, h, b1.reshape(1, D), W2)
    acc2 = _sc_msgpass(eb, g2a, g2b, zeros64)
    pa, pb = _tc_final(acc2, dinv, t, b2.reshape(1, D), We[:D].reshape(1, D),
                       We[D:].reshape(1, D), be.reshape(1, 1))

    out = _sc_edgepred(eb, pa, pb)
    return out.reshape(E, 1)


# speedup vs baseline: 32.9657x; 1.0072x over previous
import functools

import jax
import jax.numpy as jnp
from jax import lax
from jax.experimental import pallas as pl
from jax.experimental.pallas import tpu as pltpu
from jax.experimental.pallas import tpu_sc as plsc

N = 10000
E = 320000
D = 128
NC = 2
NS = 16
NW = NC * NS
EW = E // NW
B = 80
KB = EW // B
NACC = 10240
RPT = NACC // NS
NPAD = 80 * 128

_MESH = plsc.VectorSubcoreMesh(core_axis_name="c", subcore_axis_name="s")


def _wid():
    return lax.axis_index("s") * NC + lax.axis_index("c")


def _sc_deg_body(eb, zeros, out, didx, hist):
    wid = _wid()
    pltpu.sync_copy(eb.at[1, wid], didx)
    pltpu.sync_copy(zeros, hist)
    ones = jnp.full((16,), 1.0, jnp.float32)

    def step(j, carry):
        for k in range(B // 16):
            dv = didx[j, pl.ds(k * 16, 16)]
            row = lax.shift_right_logical(dv, 7)
            col = lax.bitwise_and(dv, 127)
            plsc.addupdate_scatter(hist, [row, col], ones)
        return carry

    lax.fori_loop(0, KB, step, 0)
    pltpu.sync_copy(hist, out.at[wid])


_sc_deg = pl.kernel(
    _sc_deg_body,
    out_type=jax.ShapeDtypeStruct((NW, 80, 128), jnp.float32),
    mesh=_MESH,
    compiler_params=pltpu.CompilerParams(needs_layout_passes=False),
    scratch_types=[
        pltpu.VMEM((KB, B), jnp.int32),
        pltpu.VMEM((80, 128), jnp.float32),
    ],
)


DH = D // 2


NBUF = 8


def _sc_msgpass_body(eb, g, zeros, out, sidx, didx, *rest):
    rows = rest[:NBUF]
    acc = rest[NBUF]
    gsem = rest[NBUF + 1:NBUF + 1 + NBUF]
    ssem = rest[NBUF + 1 + NBUF:]
    cid = lax.axis_index("c")
    sid = lax.axis_index("s")
    wid = sid * NC + cid
    pltpu.sync_copy(eb.at[0, wid], sidx)
    pltpu.sync_copy(eb.at[1, wid], didx)

    for half in range(2):
        if half == 1:
            def bump(j, carry):
                for k in range(B // 16):
                    sl = pl.ds(k * 16, 16)
                    sidx[j, sl] = sidx[j, sl] + 1
                return carry

            lax.fori_loop(0, KB, bump, 0)
        pltpu.sync_copy(zeros, acc.at[pl.ds(sid * RPT, RPT)])
        plsc.subcore_barrier()

        for k in range(NBUF // 2):
            pltpu.async_copy(g.at[sidx.at[k]], rows[k], gsem[k])

        def step(jj, carry):
            for k in range(NBUF):
                j = jj * NBUF + k
                b2 = (k + NBUF // 2) % NBUF

                @pl.when(j < KB)
                def _():
                    pltpu.make_async_copy(
                        g.at[sidx.at[0]], rows[k], gsem[k]).wait()
                    pltpu.async_copy(rows[k], acc.at[didx.at[j]], ssem[k],
                                     add=True)

                jn = j + NBUF // 2

                @pl.when(jn < KB)
                def _():
                    @pl.when(j >= NBUF // 2)
                    def _():
                        pltpu.make_async_copy(
                            rows[b2], acc.at[didx.at[0]], ssem[b2]).wait()
                    pltpu.async_copy(g.at[sidx.at[jn]], rows[b2], gsem[b2])

            return carry

        lax.fori_loop(0, (KB + NBUF - 1) // NBUF, step, 0)
        for k in range(NBUF):
            pltpu.make_async_copy(rows[k], acc.at[didx.at[0]],
                                  ssem[k]).wait()
        plsc.subcore_barrier()
        pltpu.sync_copy(acc.at[pl.ds(sid * RPT, RPT)],
                        out.at[cid, pl.ds(sid * RPT, RPT),
                               pl.ds(half * DH, DH)])
        plsc.subcore_barrier()


_sc_msgpass = pl.kernel(
    _sc_msgpass_body,
    out_type=jax.ShapeDtypeStruct((NC, NACC, D), jnp.float32),
    mesh=_MESH,
    compiler_params=pltpu.CompilerParams(use_tc_tiling_on_sc=False,
                                         needs_layout_passes=False),
    scratch_types=(
        [pltpu.VMEM((KB, B), jnp.int32),
         pltpu.VMEM((KB, B), jnp.int32)]
        + [pltpu.VMEM((B, DH), jnp.float32) for _ in range(NBUF)]
        + [pltpu.VMEM_SHARED((NACC, DH), jnp.float32)]
        + [pltpu.SemaphoreType.DMA for _ in range(2 * NBUF)]
    ),
)


def _sc_edgepred_body(eb, pa, pb, out, sidx, didx, pav, pbv, outv):
    wid = _wid()
    pltpu.sync_copy(eb.at[0, wid], sidx)
    pltpu.sync_copy(eb.at[1, wid], didx)
    pltpu.sync_copy(pa, pav)
    pltpu.sync_copy(pb, pbv)

    def step(j, carry):
        for k in range(B // 16):
            sl = pl.ds(k * 16, 16)
            sv = lax.shift_right_logical(sidx[j, sl], 1)
            dv = didx[j, sl]
            r = plsc.load_gather(pav, [sv]) + plsc.load_gather(pbv, [dv])
            outv[j, sl] = r
        return carry

    lax.fori_loop(0, KB, step, 0)
    pltpu.sync_copy(outv, out.at[wid])


_sc_edgepred = pl.kernel(
    _sc_edgepred_body,
    out_type=jax.ShapeDtypeStruct((NW, KB, B), jnp.float32),
    mesh=_MESH,
    compiler_params=pltpu.CompilerParams(needs_layout_passes=False),
    scratch_types=[
        pltpu.VMEM((KB, B), jnp.int32),
        pltpu.VMEM((KB, B), jnp.int32),
        pltpu.VMEM((N,), jnp.float32),
        pltpu.VMEM((N,), jnp.float32),
        pltpu.VMEM((KB, B), jnp.float32),
    ],
)


def _tc_dinv_body(parts_ref, out_ref):
    deg = jnp.sum(parts_ref[...], axis=0) + 1.0
    out_ref[...] = lax.rsqrt(deg)


def _tc_dinv(parts):
    return pl.pallas_call(
        _tc_dinv_body,
        out_shape=jax.ShapeDtypeStruct((80, 128), jnp.float32),
    )(parts)


R = 2000


def _tc_mm_body(x_ref, w_ref, dinv_ref, h_ref, g_ref):
    h = jnp.dot(x_ref[...], w_ref[...], preferred_element_type=jnp.float32)
    h_ref[...] = h
    g_ref[...] = h * dinv_ref[...]


def _tc_mm(x, w, dinv):
    return pl.pallas_call(
        _tc_mm_body,
        grid=(N // R,),
        in_specs=[
            pl.BlockSpec((R, D), lambda i: (i, 0)),
            pl.BlockSpec((D, D), lambda i: (0, 0)),
            pl.BlockSpec((R, 1), lambda i: (i, 0)),
        ],
        out_specs=[
            pl.BlockSpec((R, D), lambda i: (i, 0)),
            pl.BlockSpec((R, D), lambda i: (i, 0)),
        ],
        out_shape=[
            jax.ShapeDtypeStruct((N, D), jnp.float32),
            jax.ShapeDtypeStruct((N, D), jnp.float32),
        ],
    )(x, w, dinv)


def _tc_layer_body(parts_ref, dinv_ref, h_ref, b1_ref, w2_ref, t_ref,
                   g2_ref):
    s = parts_ref[0] + parts_ref[1]
    dv = dinv_ref[...]
    agg = dv * s + (dv * dv) * h_ref[...] + b1_ref[...]
    h1 = jnp.maximum(agg, 0.0)
    t = jnp.dot(h1, w2_ref[...], preferred_element_type=jnp.float32)
    t_ref[...] = t
    g2_ref[...] = t * dv


def _tc_layer(parts, dinv, h, b1, w2):
    return pl.pallas_call(
        _tc_layer_body,
        grid=(N // R,),
        in_specs=[
            pl.BlockSpec((NC, R, D), lambda i: (0, i, 0)),
            pl.BlockSpec((R, 1), lambda i: (i, 0)),
            pl.BlockSpec((R, D), lambda i: (i, 0)),
            pl.BlockSpec((1, D), lambda i: (0, 0)),
            pl.BlockSpec((D, D), lambda i: (0, 0)),
        ],
        out_specs=[
            pl.BlockSpec((R, D), lambda i: (i, 0)),
            pl.BlockSpec((R, D), lambda i: (i, 0)),
        ],
        out_shape=[
            jax.ShapeDtypeStruct((N, D), jnp.float32),
            jax.ShapeDtypeStruct((N, D), jnp.float32),
        ],
    )(parts, dinv, h, b1, w2)


def _tc_final_body(parts_ref, dinv_ref, t_ref, b2_ref, wa_ref, wb_ref,
                   be_ref, pa_ref, pb_ref):
    s = parts_ref[0] + parts_ref[1]
    dv = dinv_ref[...]
    h2 = dv * s + (dv * dv) * t_ref[...] + b2_ref[...]
    pa_ref[...] = jnp.sum(h2 * wa_ref[...], axis=1) + be_ref[0, 0]
    pb_ref[...] = jnp.sum(h2 * wb_ref[...], axis=1)


def _tc_final(parts, dinv, t, b2, wa, wb, be):
    return pl.pallas_call(
        _tc_final_body,
        grid=(1,),
        in_specs=[
            pl.BlockSpec((NC, N, D), lambda i: (0, 0, 0)),
            pl.BlockSpec((N, 1), lambda i: (0, 0)),
            pl.BlockSpec((N, D), lambda i: (0, 0)),
            pl.BlockSpec((1, D), lambda i: (0, 0)),
            pl.BlockSpec((1, D), lambda i: (0, 0)),
            pl.BlockSpec((1, D), lambda i: (0, 0)),
            pl.BlockSpec((1, 1), lambda i: (0, 0)),
        ],
        out_specs=[
            pl.BlockSpec((N,), lambda i: (0,)),
            pl.BlockSpec((N,), lambda i: (0,)),
        ],
        out_shape=[
            jax.ShapeDtypeStruct((N,), jnp.float32),
            jax.ShapeDtypeStruct((N,), jnp.float32),
        ],
    )(parts, dinv, t, b2, wa, wb, be)


def kernel(x, edge_index, W1, b1, W2, b2, We, be):
    eb = jnp.stack([edge_index[0] * 2, edge_index[1]]).reshape(2, NW, KB, B)
    zeros128 = jnp.zeros((80, 128), jnp.float32)
    zeros64 = jnp.zeros((RPT, DH), jnp.float32)

    deg_parts = _sc_deg(eb, zeros128)
    dinv_pad = _tc_dinv(deg_parts)
    dinv = dinv_pad.reshape(NPAD, 1)[:N]

    h, g1 = _tc_mm(x, W1, dinv)
    acc1 = _sc_msgpass(eb, g1.reshape(2 * N, DH), zeros64)
    t, g2 = _tc_layer(acc1, dinv, h, b1.reshape(1, D), W2)
    acc2 = _sc_msgpass(eb, g2.reshape(2 * N, DH), zeros64)
    pa, pb = _tc_final(acc2, dinv, t, b2.reshape(1, D), We[:D].reshape(1, D),
                       We[D:].reshape(1, D), be.reshape(1, 1))

    out = _sc_edgepred(eb, pa, pb)
    return out.reshape(E, 1)


# speedup vs baseline: 34.4261x; 1.0443x over previous
import functools

import jax
import jax.numpy as jnp
from jax import lax
from jax.experimental import pallas as pl
from jax.experimental.pallas import tpu as pltpu
from jax.experimental.pallas import tpu_sc as plsc

N = 10000
E = 320000
D = 128
NC = 2
NS = 16
NW = NC * NS
EW = E // NW
B = 80
KB = EW // B
NACC = 10240
RPT = NACC // NS
NPAD = 80 * 128

_MESH = plsc.VectorSubcoreMesh(core_axis_name="c", subcore_axis_name="s")


def _wid():
    return lax.axis_index("s") * NC + lax.axis_index("c")


def _sc_deg_body(eb, zeros, out, didx, hist):
    wid = _wid()
    pltpu.sync_copy(eb.at[1, wid], didx)
    pltpu.sync_copy(zeros, hist)
    ones = jnp.full((16,), 1.0, jnp.float32)

    def step(j, carry):
        for k in range(B // 16):
            dv = didx[j, pl.ds(k * 16, 16)]
            row = lax.shift_right_logical(dv, 7)
            col = lax.bitwise_and(dv, 127)
            plsc.addupdate_scatter(hist, [row, col], ones)
        return carry

    lax.fori_loop(0, KB, step, 0)
    pltpu.sync_copy(hist, out.at[wid])


_sc_deg = pl.kernel(
    _sc_deg_body,
    out_type=jax.ShapeDtypeStruct((NW, 80, 128), jnp.float32),
    mesh=_MESH,
    compiler_params=pltpu.CompilerParams(needs_layout_passes=False),
    scratch_types=[
        pltpu.VMEM((KB, B), jnp.int32),
        pltpu.VMEM((80, 128), jnp.float32),
    ],
)


DH = D // 2


NBUF = 8


def _sc_msgpass_body(eb, g, zeros, out, sidx, didx, *rest):
    rows = rest[:NBUF]
    acc = rest[NBUF]
    gsem = rest[NBUF + 1:NBUF + 1 + NBUF]
    ssem = rest[NBUF + 1 + NBUF:]
    cid = lax.axis_index("c")
    sid = lax.axis_index("s")
    wid = sid * NC + cid
    pltpu.sync_copy(eb.at[0, wid], sidx)
    pltpu.sync_copy(eb.at[1, wid], didx)

    for half in range(2):
        if half == 1:
            def bump(j, carry):
                for k in range(B // 16):
                    sl = pl.ds(k * 16, 16)
                    sidx[j, sl] = sidx[j, sl] + 1
                return carry

            lax.fori_loop(0, KB, bump, 0)
        pltpu.sync_copy(zeros, acc.at[pl.ds(sid * RPT, RPT)])
        plsc.subcore_barrier()

        for k in range(NBUF // 2):
            pltpu.async_copy(g.at[sidx.at[k]], rows[k], gsem[k])

        def step(jj, carry):
            for k in range(NBUF):
                j = jj * NBUF + k
                b2 = (k + NBUF // 2) % NBUF

                @pl.when(j < KB)
                def _():
                    pltpu.make_async_copy(
                        g.at[sidx.at[0]], rows[k], gsem[k]).wait()
                    pltpu.async_copy(rows[k], acc.at[didx.at[j]], ssem[k],
                                     add=True)

                jn = j + NBUF // 2

                @pl.when(jn < KB)
                def _():
                    @pl.when(j >= NBUF // 2)
                    def _():
                        pltpu.make_async_copy(
                            rows[b2], acc.at[didx.at[0]], ssem[b2]).wait()
                    pltpu.async_copy(g.at[sidx.at[jn]], rows[b2], gsem[b2])

            return carry

        lax.fori_loop(0, (KB + NBUF - 1) // NBUF, step, 0)
        for k in range(NBUF):
            pltpu.make_async_copy(rows[k], acc.at[didx.at[0]],
                                  ssem[k]).wait()
        plsc.subcore_barrier()
        pltpu.sync_copy(acc.at[pl.ds(sid * RPT, RPT)],
                        out.at[cid, pl.ds(sid * RPT, RPT),
                               pl.ds(half * DH, DH)])
        plsc.subcore_barrier()


_sc_msgpass = pl.kernel(
    _sc_msgpass_body,
    out_type=jax.ShapeDtypeStruct((NC, NACC, D), jnp.float32),
    mesh=_MESH,
    compiler_params=pltpu.CompilerParams(use_tc_tiling_on_sc=False,
                                         needs_layout_passes=False),
    scratch_types=(
        [pltpu.VMEM((KB, B), jnp.int32),
         pltpu.VMEM((KB, B), jnp.int32)]
        + [pltpu.VMEM((B, DH), jnp.float32) for _ in range(NBUF)]
        + [pltpu.VMEM_SHARED((NACC, DH), jnp.float32)]
        + [pltpu.SemaphoreType.DMA for _ in range(2 * NBUF)]
    ),
)


def _sc_edgepred_body(eb, pa, pb, out, sidx, didx, pav, pbv, outv):
    wid = _wid()
    pltpu.sync_copy(eb.at[0, wid], sidx)
    pltpu.sync_copy(eb.at[1, wid], didx)
    pltpu.sync_copy(pa, pav)
    pltpu.sync_copy(pb, pbv)

    def step(j, carry):
        for k in range(B // 16):
            sl = pl.ds(k * 16, 16)
            sv = lax.shift_right_logical(sidx[j, sl], 1)
            dv = didx[j, sl]
            r = plsc.load_gather(pav, [sv]) + plsc.load_gather(pbv, [dv])
            outv[j, sl] = r
        return carry

    lax.fori_loop(0, KB, step, 0)
    pltpu.sync_copy(outv, out.at[wid])


_sc_edgepred = pl.kernel(
    _sc_edgepred_body,
    out_type=jax.ShapeDtypeStruct((NW, KB, B), jnp.float32),
    mesh=_MESH,
    compiler_params=pltpu.CompilerParams(needs_layout_passes=False),
    scratch_types=[
        pltpu.VMEM((KB, B), jnp.int32),
        pltpu.VMEM((KB, B), jnp.int32),
        pltpu.VMEM((N,), jnp.float32),
        pltpu.VMEM((N,), jnp.float32),
        pltpu.VMEM((KB, B), jnp.float32),
    ],
)


def _tc_dinv_body(parts_ref, out_ref):
    deg = jnp.sum(parts_ref[...], axis=0) + 1.0
    out_ref[...] = lax.rsqrt(deg)


def _tc_dinv(parts):
    return pl.pallas_call(
        _tc_dinv_body,
        out_shape=jax.ShapeDtypeStruct((80, 128), jnp.float32),
    )(parts)


R = 2000


def _tc_mm_body(x_ref, w_ref, dinv_ref, h_ref, g_ref):
    h = jnp.dot(x_ref[...], w_ref[...], preferred_element_type=jnp.float32)
    h_ref[...] = h
    g_ref[...] = h * dinv_ref[...]


def _tc_mm(x, w, dinv):
    return pl.pallas_call(
        _tc_mm_body,
        grid=(N // R,),
        in_specs=[
            pl.BlockSpec((R, D), lambda i: (i, 0)),
            pl.BlockSpec((D, D), lambda i: (0, 0)),
            pl.BlockSpec((R, 1), lambda i: (i, 0)),
        ],
        out_specs=[
            pl.BlockSpec((R, D), lambda i: (i, 0)),
            pl.BlockSpec((R, D), lambda i: (i, 0)),
        ],
        out_shape=[
            jax.ShapeDtypeStruct((N, D), jnp.float32),
            jax.ShapeDtypeStruct((N, D), jnp.float32),
        ],
    )(x, w, dinv)


def _tc_layer_body(parts_ref, dinv_ref, h_ref, b1_ref, w2_ref, t_ref,
                   g2_ref):
    s = parts_ref[0] + parts_ref[1]
    dv = dinv_ref[...]
    agg = dv * s + (dv * dv) * h_ref[...] + b1_ref[...]
    h1 = jnp.maximum(agg, 0.0)
    t = jnp.dot(h1, w2_ref[...], preferred_element_type=jnp.float32)
    t_ref[...] = t
    g2_ref[...] = t * dv


def _tc_layer(parts, dinv, h, b1, w2):
    return pl.pallas_call(
        _tc_layer_body,
        grid=(N // R,),
        in_specs=[
            pl.BlockSpec((NC, R, D), lambda i: (0, i, 0)),
            pl.BlockSpec((R, 1), lambda i: (i, 0)),
            pl.BlockSpec((R, D), lambda i: (i, 0)),
            pl.BlockSpec((1, D), lambda i: (0, 0)),
            pl.BlockSpec((D, D), lambda i: (0, 0)),
        ],
        out_specs=[
            pl.BlockSpec((R, D), lambda i: (i, 0)),
            pl.BlockSpec((R, D), lambda i: (i, 0)),
        ],
        out_shape=[
            jax.ShapeDtypeStruct((N, D), jnp.float32),
            jax.ShapeDtypeStruct((N, D), jnp.float32),
        ],
    )(parts, dinv, h, b1, w2)


def _tc_final_body(parts_ref, dinv_ref, t_ref, b2_ref, wa_ref, wb_ref,
                   be_ref, pa_ref, pb_ref):
    s = parts_ref[0] + parts_ref[1]
    dv = dinv_ref[...]
    h2 = dv * s + (dv * dv) * t_ref[...] + b2_ref[...]
    pa_ref[...] = jnp.sum(h2 * wa_ref[...], axis=1) + be_ref[0, 0]
    pb_ref[...] = jnp.sum(h2 * wb_ref[...], axis=1)


def _tc_final(parts, dinv, t, b2, wa, wb, be):
    return pl.pallas_call(
        _tc_final_body,
        grid=(1,),
        in_specs=[
            pl.BlockSpec((NC, N, D), lambda i: (0, 0, 0)),
            pl.BlockSpec((N, 1), lambda i: (0, 0)),
            pl.BlockSpec((N, D), lambda i: (0, 0)),
            pl.BlockSpec((1, D), lambda i: (0, 0)),
            pl.BlockSpec((1, D), lambda i: (0, 0)),
            pl.BlockSpec((1, D), lambda i: (0, 0)),
            pl.BlockSpec((1, 1), lambda i: (0, 0)),
        ],
        out_specs=[
            pl.BlockSpec((N,), lambda i: (0,)),
            pl.BlockSpec((N,), lambda i: (0,)),
        ],
        out_shape=[
            jax.ShapeDtypeStruct((N,), jnp.float32),
            jax.ShapeDtypeStruct((N,), jnp.float32),
        ],
    )(parts, dinv, t, b2, wa, wb, be)


def kernel(x, edge_index, W1, b1, W2, b2, We, be):
    eb = (edge_index * jnp.array([2, 1], edge_index.dtype)[:, None]
          ).reshape(2, NW, KB, B)
    zeros128 = jnp.zeros((80, 128), jnp.float32)
    zeros64 = jnp.zeros((RPT, DH), jnp.float32)

    deg_parts = _sc_deg(eb, zeros128)
    dinv_pad = _tc_dinv(deg_parts)
    dinv = dinv_pad.reshape(NPAD, 1)[:N]

    h, g1 = _tc_mm(x, W1, dinv)
    acc1 = _sc_msgpass(eb, g1.reshape(2 * N, DH), zeros64)
    t, g2 = _tc_layer(acc1, dinv, h, b1.reshape(1, D), W2)
    acc2 = _sc_msgpass(eb, g2.reshape(2 * N, DH), zeros64)
    pa, pb = _tc_final(acc2, dinv, t, b2.reshape(1, D), We[:D].reshape(1, D),
                       We[D:].reshape(1, D), be.reshape(1, 1))

    out = _sc_edgepred(eb, pa, pb)
    return out.reshape(E, 1)


# speedup vs baseline: 34.8532x; 1.0124x over previous
import functools

import jax
import jax.numpy as jnp
from jax import lax
from jax.experimental import pallas as pl
from jax.experimental.pallas import tpu as pltpu
from jax.experimental.pallas import tpu_sc as plsc

N = 10000
E = 320000
D = 128
NC = 2
NS = 16
NW = NC * NS
EW = E // NW
B = 80
KB = EW // B
NACC = 10240
RPT = NACC // NS
NPAD = 80 * 128

_MESH = plsc.VectorSubcoreMesh(core_axis_name="c", subcore_axis_name="s")


def _wid():
    return lax.axis_index("s") * NC + lax.axis_index("c")


def _sc_deg_body(eb, zeros, out, didx, hist):
    wid = _wid()
    pltpu.sync_copy(eb.at[1, wid], didx)
    pltpu.sync_copy(zeros, hist)
    ones = jnp.full((16,), 1.0, jnp.float32)

    def step(j, carry):
        for k in range(B // 16):
            dv = didx[j, pl.ds(k * 16, 16)]
            row = lax.shift_right_logical(dv, 7)
            col = lax.bitwise_and(dv, 127)
            plsc.addupdate_scatter(hist, [row, col], ones)
        return carry

    lax.fori_loop(0, KB, step, 0)
    pltpu.sync_copy(hist, out.at[wid])


_sc_deg = pl.kernel(
    _sc_deg_body,
    out_type=jax.ShapeDtypeStruct((NW, 80, 128), jnp.float32),
    mesh=_MESH,
    compiler_params=pltpu.CompilerParams(needs_layout_passes=False),
    scratch_types=[
        pltpu.VMEM((KB, B), jnp.int32),
        pltpu.VMEM((80, 128), jnp.float32),
    ],
)


DH = D // 2


NBUF = 8


def _sc_msgpass_body(eb, g, zeros, out, sidx, didx, *rest):
    rows = rest[:NBUF]
    acc = rest[NBUF]
    gsem = rest[NBUF + 1:NBUF + 1 + NBUF]
    ssem = rest[NBUF + 1 + NBUF:]
    cid = lax.axis_index("c")
    sid = lax.axis_index("s")
    wid = sid * NC + cid
    pltpu.sync_copy(eb.at[0, wid], sidx)
    pltpu.sync_copy(eb.at[1, wid], didx)

    for half in range(2):
        if half == 1:
            def bump(j, carry):
                for k in range(B // 16):
                    sl = pl.ds(k * 16, 16)
                    sidx[j, sl] = sidx[j, sl] + 1
                return carry

            lax.fori_loop(0, KB, bump, 0)
        pltpu.sync_copy(zeros, acc.at[pl.ds(sid * RPT, RPT)])
        plsc.subcore_barrier()

        for k in range(NBUF // 2):
            pltpu.async_copy(g.at[sidx.at[k]], rows[k], gsem[k])

        def step(jj, carry):
            for k in range(NBUF):
                j = jj * NBUF + k
                b2 = (k + NBUF // 2) % NBUF

                @pl.when(j < KB)
                def _():
                    pltpu.make_async_copy(
                        g.at[sidx.at[0]], rows[k], gsem[k]).wait()
                    pltpu.async_copy(rows[k], acc.at[didx.at[j]], ssem[k],
                                     add=True)

                jn = j + NBUF // 2

                @pl.when(jn < KB)
                def _():
                    @pl.when(j >= NBUF // 2)
                    def _():
                        pltpu.make_async_copy(
                            rows[b2], acc.at[didx.at[0]], ssem[b2]).wait()
                    pltpu.async_copy(g.at[sidx.at[jn]], rows[b2], gsem[b2])

            return carry

        lax.fori_loop(0, (KB + NBUF - 1) // NBUF, step, 0)
        for k in range(NBUF):
            pltpu.make_async_copy(rows[k], acc.at[didx.at[0]],
                                  ssem[k]).wait()
        plsc.subcore_barrier()
        pltpu.sync_copy(acc.at[pl.ds(sid * RPT, RPT)],
                        out.at[cid, pl.ds(sid * RPT, RPT),
                               pl.ds(half * DH, DH)])
        plsc.subcore_barrier()


_sc_msgpass = pl.kernel(
    _sc_msgpass_body,
    out_type=jax.ShapeDtypeStruct((NC, NACC, D), jnp.float32),
    mesh=_MESH,
    compiler_params=pltpu.CompilerParams(use_tc_tiling_on_sc=False,
                                         needs_layout_passes=False),
    scratch_types=(
        [pltpu.VMEM((KB, B), jnp.int32),
         pltpu.VMEM((KB, B), jnp.int32)]
        + [pltpu.VMEM((B, DH), jnp.float32) for _ in range(NBUF)]
        + [pltpu.VMEM_SHARED((NACC, DH), jnp.float32)]
        + [pltpu.SemaphoreType.DMA for _ in range(2 * NBUF)]
    ),
)


def _sc_edgepred_body(eb, pa, pb, out, sidx, didx, pav, pbv, outv):
    wid = _wid()
    pltpu.sync_copy(eb.at[0, wid], sidx)
    pltpu.sync_copy(eb.at[1, wid], didx)
    pltpu.sync_copy(pa.at[pl.ds(0, N)], pav)
    pltpu.sync_copy(pb.at[pl.ds(0, N)], pbv)

    def step(j, carry):
        for k in range(B // 16):
            sl = pl.ds(k * 16, 16)
            sv = lax.shift_right_logical(sidx[j, sl], 1)
            dv = didx[j, sl]
            r = plsc.load_gather(pav, [sv]) + plsc.load_gather(pbv, [dv])
            outv[j, sl] = r
        return carry

    lax.fori_loop(0, KB, step, 0)
    pltpu.sync_copy(outv, out.at[wid])


_sc_edgepred = pl.kernel(
    _sc_edgepred_body,
    out_type=jax.ShapeDtypeStruct((NW, KB, B), jnp.float32),
    mesh=_MESH,
    compiler_params=pltpu.CompilerParams(needs_layout_passes=False),
    scratch_types=[
        pltpu.VMEM((KB, B), jnp.int32),
        pltpu.VMEM((KB, B), jnp.int32),
        pltpu.VMEM((N,), jnp.float32),
        pltpu.VMEM((N,), jnp.float32),
        pltpu.VMEM((KB, B), jnp.float32),
    ],
)


def _tc_dinv_body(parts_ref, out_ref):
    deg = jnp.sum(parts_ref[...], axis=0) + 1.0
    out_ref[...] = lax.rsqrt(deg)


def _tc_dinv(parts):
    return pl.pallas_call(
        _tc_dinv_body,
        out_shape=jax.ShapeDtypeStruct((80, 128), jnp.float32),
    )(parts)


R = 2000


def _tc_mm_body(x_ref, w_ref, dinv_ref, h_ref, g_ref):
    h = jnp.dot(x_ref[...], w_ref[...], preferred_element_type=jnp.float32)
    h_ref[...] = h
    g_ref[...] = h * dinv_ref[...]


def _tc_mm(x, w, dinv):
    return pl.pallas_call(
        _tc_mm_body,
        grid=(N // R,),
        in_specs=[
            pl.BlockSpec((R, D), lambda i: (i, 0)),
            pl.BlockSpec((D, D), lambda i: (0, 0)),
            pl.BlockSpec((R, 1), lambda i: (i, 0)),
        ],
        out_specs=[
            pl.BlockSpec((R, D), lambda i: (i, 0)),
            pl.BlockSpec((R, D), lambda i: (i, 0)),
        ],
        out_shape=[
            jax.ShapeDtypeStruct((N, D), jnp.float32),
            jax.ShapeDtypeStruct((N, D), jnp.float32),
        ],
    )(x, w, dinv)


def _tc_layer_body(parts_ref, dinv_ref, h_ref, b1_ref, w2_ref, t_ref,
                   g2_ref):
    s = parts_ref[0] + parts_ref[1]
    dv = dinv_ref[...]
    agg = dv * s + (dv * dv) * h_ref[...] + b1_ref[...]
    h1 = jnp.maximum(agg, 0.0)
    t = jnp.dot(h1, w2_ref[...], preferred_element_type=jnp.float32)
    t_ref[...] = t
    g2_ref[...] = t * dv


def _tc_layer(parts, dinv, h, b1, w2):
    return pl.pallas_call(
        _tc_layer_body,
        grid=(N // R,),
        in_specs=[
            pl.BlockSpec((NC, R, D), lambda i: (0, i, 0)),
            pl.BlockSpec((R, 1), lambda i: (i, 0)),
            pl.BlockSpec((R, D), lambda i: (i, 0)),
            pl.BlockSpec((1, D), lambda i: (0, 0)),
            pl.BlockSpec((D, D), lambda i: (0, 0)),
        ],
        out_specs=[
            pl.BlockSpec((R, D), lambda i: (i, 0)),
            pl.BlockSpec((R, D), lambda i: (i, 0)),
        ],
        out_shape=[
            jax.ShapeDtypeStruct((N, D), jnp.float32),
            jax.ShapeDtypeStruct((N, D), jnp.float32),
        ],
    )(parts, dinv, h, b1, w2)


def _tc_final_body(parts_ref, dinv_ref, t_ref, b2_ref, wa_ref, wb_ref,
                   be_ref, pa_ref, pb_ref):
    s = parts_ref[0] + parts_ref[1]
    dv = dinv_ref[...]
    h2 = dv * s + (dv * dv) * t_ref[...] + b2_ref[...]
    pa_ref[...] = jnp.sum(h2 * wa_ref[...], axis=1) + be_ref[0, 0]
    pb_ref[...] = jnp.sum(h2 * wb_ref[...], axis=1)


def _tc_final(parts, dinv, t, b2, wa, wb, be):
    RF = 1024
    return pl.pallas_call(
        _tc_final_body,
        grid=(NACC // RF,),
        in_specs=[
            pl.BlockSpec((NC, RF, D), lambda i: (0, i, 0)),
            pl.BlockSpec((RF, 1), lambda i: (i, 0)),
            pl.BlockSpec((RF, D), lambda i: (i, 0)),
            pl.BlockSpec((1, D), lambda i: (0, 0)),
            pl.BlockSpec((1, D), lambda i: (0, 0)),
            pl.BlockSpec((1, D), lambda i: (0, 0)),
            pl.BlockSpec((1, 1), lambda i: (0, 0)),
        ],
        out_specs=[
            pl.BlockSpec((RF,), lambda i: (i,)),
            pl.BlockSpec((RF,), lambda i: (i,)),
        ],
        out_shape=[
            jax.ShapeDtypeStruct((NACC,), jnp.float32),
            jax.ShapeDtypeStruct((NACC,), jnp.float32),
        ],
    )(parts, dinv, t, b2, wa, wb, be)


def kernel(x, edge_index, W1, b1, W2, b2, We, be):
    eb = (edge_index * jnp.array([2, 1], edge_index.dtype)[:, None]
          ).reshape(2, NW, KB, B)
    zeros128 = jnp.zeros((80, 128), jnp.float32)
    zeros64 = jnp.zeros((RPT, DH), jnp.float32)

    deg_parts = _sc_deg(eb, zeros128)
    dinv_pad = _tc_dinv(deg_parts)
    dinv = dinv_pad.reshape(NPAD, 1)

    h, g1 = _tc_mm(x, W1, dinv)
    acc1 = _sc_msgpass(eb, g1.reshape(2 * N, DH), zeros64)
    t, g2 = _tc_layer(acc1, dinv, h, b1.reshape(1, D), W2)
    acc2 = _sc_msgpass(eb, g2.reshape(2 * N, DH), zeros64)
    pa, pb = _tc_final(acc2, dinv, t, b2.reshape(1, D), We[:D].reshape(1, D),
                       We[D:].reshape(1, D), be.reshape(1, 1))

    out = _sc_edgepred(eb, pa, pb)
    return out.reshape(E, 1)


# speedup vs baseline: 34.8964x; 1.0012x over previous
import functools

import jax
import jax.numpy as jnp
from jax import lax
from jax.experimental import pallas as pl
from jax.experimental.pallas import tpu as pltpu
from jax.experimental.pallas import tpu_sc as plsc

N = 10000
E = 320000
D = 128
NC = 2
NS = 16
NW = NC * NS
EW = E // NW
B = 80
KB = EW // B
NACC = 10240
RPT = NACC // NS
NPAD = 80 * 128

_MESH = plsc.VectorSubcoreMesh(core_axis_name="c", subcore_axis_name="s")


def _wid():
    return lax.axis_index("s") * NC + lax.axis_index("c")


def _sc_deg_body(eb, zeros, out, didx, hist):
    wid = _wid()
    pltpu.sync_copy(eb.at[1, wid], didx)
    pltpu.sync_copy(zeros, hist)
    ones = jnp.full((16,), 1.0, jnp.float32)

    def step(j, carry):
        for k in range(B // 16):
            dv = didx[j, pl.ds(k * 16, 16)]
            row = lax.shift_right_logical(dv, 7)
            col = lax.bitwise_and(dv, 127)
            plsc.addupdate_scatter(hist, [row, col], ones)
        return carry

    lax.fori_loop(0, KB, step, 0)
    pltpu.sync_copy(hist, out.at[wid])


_sc_deg = pl.kernel(
    _sc_deg_body,
    out_type=jax.ShapeDtypeStruct((NW, 80, 128), jnp.float32),
    mesh=_MESH,
    compiler_params=pltpu.CompilerParams(needs_layout_passes=False),
    scratch_types=[
        pltpu.VMEM((KB, B), jnp.int32),
        pltpu.VMEM((80, 128), jnp.float32),
    ],
)


DH = D // 2


NBUF = 8


def _sc_msgpass_body(eb, g, zeros, out, sidx, didx, *rest):
    rows = rest[:NBUF]
    acc = rest[NBUF]
    gsem = rest[NBUF + 1:NBUF + 1 + NBUF]
    ssem = rest[NBUF + 1 + NBUF:]
    cid = lax.axis_index("c")
    sid = lax.axis_index("s")
    wid = sid * NC + cid
    pltpu.sync_copy(eb.at[0, wid], sidx)
    pltpu.sync_copy(eb.at[1, wid], didx)

    for half in range(2):
        if half == 1:
            def bump(j, carry):
                for k in range(B // 16):
                    sl = pl.ds(k * 16, 16)
                    sidx[j, sl] = sidx[j, sl] + 1
                return carry

            lax.fori_loop(0, KB, bump, 0)
        pltpu.sync_copy(zeros, acc.at[pl.ds(sid * RPT, RPT)])
        plsc.subcore_barrier()

        for k in range(NBUF // 2):
            pltpu.async_copy(g.at[sidx.at[k]], rows[k], gsem[k])

        def step(jj, carry):
            for k in range(NBUF):
                j = jj * NBUF + k
                b2 = (k + NBUF // 2) % NBUF

                @pl.when(j < KB)
                def _():
                    pltpu.make_async_copy(
                        g.at[sidx.at[0]], rows[k], gsem[k]).wait()
                    pltpu.async_copy(rows[k], acc.at[didx.at[j]], ssem[k],
                                     add=True)

                jn = j + NBUF // 2

                @pl.when(jn < KB)
                def _():
                    @pl.when(j >= NBUF // 2)
                    def _():
                        pltpu.make_async_copy(
                            rows[b2], acc.at[didx.at[0]], ssem[b2]).wait()
                    pltpu.async_copy(g.at[sidx.at[jn]], rows[b2], gsem[b2])

            return carry

        lax.fori_loop(0, (KB + NBUF - 1) // NBUF, step, 0)
        for k in range(NBUF):
            pltpu.make_async_copy(rows[k], acc.at[didx.at[0]],
                                  ssem[k]).wait()
        plsc.subcore_barrier()
        pltpu.sync_copy(acc.at[pl.ds(sid * RPT, RPT)],
                        out.at[cid, pl.ds(sid * RPT, RPT),
                               pl.ds(half * DH, DH)])
        plsc.subcore_barrier()


_sc_msgpass = pl.kernel(
    _sc_msgpass_body,
    out_type=jax.ShapeDtypeStruct((NC, NACC, D), jnp.float32),
    mesh=_MESH,
    compiler_params=pltpu.CompilerParams(use_tc_tiling_on_sc=False,
                                         needs_layout_passes=False),
    scratch_types=(
        [pltpu.VMEM((KB, B), jnp.int32),
         pltpu.VMEM((KB, B), jnp.int32)]
        + [pltpu.VMEM((B, DH), jnp.float32) for _ in range(NBUF)]
        + [pltpu.VMEM_SHARED((NACC, DH), jnp.float32)]
        + [pltpu.SemaphoreType.DMA for _ in range(2 * NBUF)]
    ),
)


def _sc_edgepred_body(eb, pa, pb, out, sidx, didx, pav, pbv, outv):
    wid = _wid()
    pltpu.sync_copy(eb.at[0, wid], sidx)
    pltpu.sync_copy(eb.at[1, wid], didx)
    pltpu.sync_copy(pa.at[pl.ds(0, N)], pav)
    pltpu.sync_copy(pb.at[pl.ds(0, N)], pbv)

    def step(j, carry):
        for k in range(B // 16):
            sl = pl.ds(k * 16, 16)
            sv = lax.shift_right_logical(sidx[j, sl], 1)
            dv = didx[j, sl]
            r = plsc.load_gather(pav, [sv]) + plsc.load_gather(pbv, [dv])
            outv[j, sl] = r
        return carry

    lax.fori_loop(0, KB, step, 0)
    pltpu.sync_copy(outv, out.at[wid])


_sc_edgepred = pl.kernel(
    _sc_edgepred_body,
    out_type=jax.ShapeDtypeStruct((NW, KB, B), jnp.float32),
    mesh=_MESH,
    compiler_params=pltpu.CompilerParams(needs_layout_passes=False),
    scratch_types=[
        pltpu.VMEM((KB, B), jnp.int32),
        pltpu.VMEM((KB, B), jnp.int32),
        pltpu.VMEM((N,), jnp.float32),
        pltpu.VMEM((N,), jnp.float32),
        pltpu.VMEM((KB, B), jnp.float32),
    ],
)


def _tc_dinv_body(parts_ref, out_ref):
    deg = jnp.sum(parts_ref[...], axis=0) + 1.0
    out_ref[...] = 1.0 / jnp.sqrt(deg)


def _tc_dinv(parts):
    return pl.pallas_call(
        _tc_dinv_body,
        out_shape=jax.ShapeDtypeStruct((80, 128), jnp.float32),
    )(parts)


R = 2000


def _tc_mm_body(x_ref, w_ref, dinv_ref, h_ref, g_ref):
    h = jnp.dot(x_ref[...], w_ref[...], preferred_element_type=jnp.float32)
    h_ref[...] = h
    g_ref[...] = h * dinv_ref[...]


def _tc_mm(x, w, dinv):
    return pl.pallas_call(
        _tc_mm_body,
        grid=(N // R,),
        in_specs=[
            pl.BlockSpec((R, D), lambda i: (i, 0)),
            pl.BlockSpec((D, D), lambda i: (0, 0)),
            pl.BlockSpec((R, 1), lambda i: (i, 0)),
        ],
        out_specs=[
            pl.BlockSpec((R, D), lambda i: (i, 0)),
            pl.BlockSpec((R, D), lambda i: (i, 0)),
        ],
        out_shape=[
            jax.ShapeDtypeStruct((N, D), jnp.float32),
            jax.ShapeDtypeStruct((N, D), jnp.float32),
        ],
    )(x, w, dinv)


def _tc_layer_body(parts_ref, dinv_ref, h_ref, b1_ref, w2_ref, t_ref,
                   g2_ref):
    s = parts_ref[0] + parts_ref[1]
    dv = dinv_ref[...]
    agg = dv * s + (dv * dv) * h_ref[...] + b1_ref[...]
    h1 = jnp.maximum(agg, 0.0)
    t = jnp.dot(h1, w2_ref[...], preferred_element_type=jnp.float32)
    t_ref[...] = t
    g2_ref[...] = t * dv


def _tc_layer(parts, dinv, h, b1, w2):
    return pl.pallas_call(
        _tc_layer_body,
        grid=(N // R,),
        in_specs=[
            pl.BlockSpec((NC, R, D), lambda i: (0, i, 0)),
            pl.BlockSpec((R, 1), lambda i: (i, 0)),
            pl.BlockSpec((R, D), lambda i: (i, 0)),
            pl.BlockSpec((1, D), lambda i: (0, 0)),
            pl.BlockSpec((D, D), lambda i: (0, 0)),
        ],
        out_specs=[
            pl.BlockSpec((R, D), lambda i: (i, 0)),
            pl.BlockSpec((R, D), lambda i: (i, 0)),
        ],
        out_shape=[
            jax.ShapeDtypeStruct((N, D), jnp.float32),
            jax.ShapeDtypeStruct((N, D), jnp.float32),
        ],
    )(parts, dinv, h, b1, w2)


def _tc_final_body(parts_ref, dinv_ref, t_ref, b2_ref, wa_ref, wb_ref,
                   be_ref, pa_ref, pb_ref):
    s = parts_ref[0] + parts_ref[1]
    dv = dinv_ref[...]
    h2 = dv * s + (dv * dv) * t_ref[...] + b2_ref[...]
    h2b = h2.astype(jnp.bfloat16).astype(jnp.float32)
    wab = wa_ref[...].astype(jnp.bfloat16).astype(jnp.float32)
    wbb = wb_ref[...].astype(jnp.bfloat16).astype(jnp.float32)
    pa_ref[...] = jnp.sum(h2b * wab, axis=1) + be_ref[0, 0]
    pb_ref[...] = jnp.sum(h2b * wbb, axis=1)


def _tc_final(parts, dinv, t, b2, wa, wb, be):
    RF = 1024
    return pl.pallas_call(
        _tc_final_body,
        grid=(NACC // RF,),
        in_specs=[
            pl.BlockSpec((NC, RF, D), lambda i: (0, i, 0)),
            pl.BlockSpec((RF, 1), lambda i: (i, 0)),
            pl.BlockSpec((RF, D), lambda i: (i, 0)),
            pl.BlockSpec((1, D), lambda i: (0, 0)),
            pl.BlockSpec((1, D), lambda i: (0, 0)),
            pl.BlockSpec((1, D), lambda i: (0, 0)),
            pl.BlockSpec((1, 1), lambda i: (0, 0)),
        ],
        out_specs=[
            pl.BlockSpec((RF,), lambda i: (i,)),
            pl.BlockSpec((RF,), lambda i: (i,)),
        ],
        out_shape=[
            jax.ShapeDtypeStruct((NACC,), jnp.float32),
            jax.ShapeDtypeStruct((NACC,), jnp.float32),
        ],
    )(parts, dinv, t, b2, wa, wb, be)


def kernel(x, edge_index, W1, b1, W2, b2, We, be):
    eb = (edge_index * jnp.array([2, 1], edge_index.dtype)[:, None]
          ).reshape(2, NW, KB, B)
    zeros128 = jnp.zeros((80, 128), jnp.float32)
    zeros64 = jnp.zeros((RPT, DH), jnp.float32)

    deg_parts = _sc_deg(eb, zeros128)
    dinv_pad = _tc_dinv(deg_parts)
    dinv = dinv_pad.reshape(NPAD, 1)

    h, g1 = _tc_mm(x, W1, dinv)
    acc1 = _sc_msgpass(eb, g1.reshape(2 * N, DH), zeros64)
    t, g2 = _tc_layer(acc1, dinv, h, b1.reshape(1, D), W2)
    acc2 = _sc_msgpass(eb, g2.reshape(2 * N, DH), zeros64)
    pa, pb = _tc_final(acc2, dinv, t, b2.reshape(1, D), We[:D].reshape(1, D),
                       We[D:].reshape(1, D), be.reshape(1, 1))

    out = _sc_edgepred(eb, pa, pb)
    return out.reshape(E, 1)
